# baseline (speedup 1.0000x reference)
import functools

import jax
import jax.numpy as jnp
from jax import lax
from jax.experimental import pallas as pl
from jax.experimental.pallas import tpu as pltpu
from jax.experimental.pallas import tpu_sc as plsc

D = 128
OBJ_D = 96
ATTR_D = 32
ANG_D = 32
BOX_D = 96
GD = 2 * D
H = 4 * D
NANGLE = 24
NV = 10000
NVP = 10240
NE = 160000
NW = 32

_F32 = jnp.float32
_I32 = jnp.int32


def _sc_mesh():
    return plsc.VectorSubcoreMesh(
        core_axis_name="c", subcore_axis_name="s", num_cores=2, num_subcores=16)


def _pipelined_n(ntasks, nbuf, start, consume):
    for t in range(min(nbuf - 1, ntasks)):
        start(t, t % nbuf)
    ngroups = ntasks // nbuf
    if ngroups:
        @pl.loop(0, nbuf * ngroups, step=nbuf)
        def _(j):
            for b in range(nbuf):
                t = j + b

                @pl.when(t + nbuf - 1 < ntasks)
                def _():
                    start(t + nbuf - 1, (b + nbuf - 1) % nbuf)

                consume(t, b)
    for t in range(nbuf * ngroups, ntasks):
        consume(t, t % nbuf)


def _pipelined(nfull, start, consume):
    if nfull <= 0:
        return
    start(0, 0)
    npairs = nfull // 2
    if npairs:
        @pl.loop(0, 2 * npairs, step=2)
        def _(j):
            for b in range(2):
                jj = j + b

                @pl.when(jj + 1 < nfull)
                def _():
                    start(jj + 1, 1 - b)

                consume(jj, b)
    if nfull % 2:
        consume(nfull - 1, (nfull - 1) % 2)


def _gather_stream(table_h, idx_h, out_h, bufs, base, nfull, rem):
    (idx_v, rows_v, sems) = bufs

    def start(j, b):
        off = pl.multiple_of(base + j * 128, 8)
        pltpu.sync_copy(idx_h.at[pl.ds(off, 128)], idx_v[b])
        pltpu.async_copy(table_h.at[idx_v[b]], rows_v[b], sems[b])

    def consume(j, b):
        off = pl.multiple_of(base + j * 128, 8)
        pltpu.make_async_copy(
            table_h.at[idx_v[b]], rows_v[b], sems[b]).wait()
        pltpu.sync_copy(rows_v[b], out_h.at[pl.ds(off, 128)])

    _pipelined(nfull, start, consume)

    if rem:
        off = pl.multiple_of(base + nfull * 128, 8)
        pltpu.sync_copy(idx_h.at[pl.ds(off, rem)], idx_v[2])
        pltpu.async_copy(table_h.at[idx_v[2]], rows_v[2], sems[0]).wait()
        pltpu.sync_copy(rows_v[2], out_h.at[pl.ds(off, rem)])


def _gather_scratch(row_shape, dtype, rem):
    sc = [
        pltpu.VMEM((128,), _I32), pltpu.VMEM((128,), _I32),
        pltpu.VMEM((max(rem, 8),), _I32),
        pltpu.VMEM((128,) + row_shape, dtype),
        pltpu.VMEM((128,) + row_shape, dtype),
        pltpu.VMEM((max(rem, 8),) + row_shape, dtype),
        pltpu.SemaphoreType.DMA, pltpu.SemaphoreType.DMA,
    ]
    return sc


def _sc_gather(table, idx, name):
    B = idx.shape[0]
    row_shape = table.shape[1:]
    bpw = B // NW
    assert B % NW == 0 and bpw % 8 == 0
    nfull, rem = divmod(bpw, 128)

    @functools.partial(
        pl.kernel,
        out_type=jax.ShapeDtypeStruct((B,) + row_shape, table.dtype),
        mesh=_sc_mesh(),
        scratch_types=_gather_scratch(row_shape, table.dtype, rem),
        name=name,
    )
    def k(table_h, idx_h, out_h, i0, i1, i2, r0, r1, r2, s0, s1):
        wid = lax.axis_index("s") * 2 + lax.axis_index("c")
        base = pl.multiple_of(wid * bpw, 8)
        bufs = ((i0, i1, i2), (r0, r1, r2), (s0, s1))
        _gather_stream(table_h, idx_h, out_h, bufs, base, nfull, rem)

    return k(table, idx)


def _sc_gather2(table, idx_s, idx_o, name):
    B = idx_s.shape[0]
    row_shape = table.shape[1:]
    bpw = B // NW
    assert B % NW == 0 and bpw % 8 == 0
    nfull, rem = divmod(bpw, 128)
    out = jax.ShapeDtypeStruct((B,) + row_shape, table.dtype)
    scratch = (
        [pltpu.VMEM((128,), _I32)] * 4
        + [pltpu.VMEM((max(rem, 8),), _I32)]
        + [pltpu.VMEM((128,) + row_shape, table.dtype)] * 4
        + [pltpu.VMEM((max(rem, 8),) + row_shape, table.dtype)]
        + [pltpu.SemaphoreType.DMA] * 4)

    @functools.partial(
        pl.kernel,
        out_type=[out, out],
        mesh=_sc_mesh(),
        scratch_types=scratch,
        name=name,
    )
    def k(table_h, is_h, io_h, os_h, oo_h,
          i0, i1, i2, i3, ir, r0, r1, r2, r3, rr, s0, s1, s2, s3):
        wid = lax.axis_index("s") * 2 + lax.axis_index("c")
        base = pl.multiple_of(wid * bpw, 8)
        idx_v = (i0, i1, i2, i3)
        rows_v = (r0, r1, r2, r3)
        sems = (s0, s1, s2, s3)

        def start(t, b):
            ii_h = is_h if b % 2 == 0 else io_h
            off = pl.multiple_of(base + (t // 2) * 128, 8)
            pltpu.sync_copy(ii_h.at[pl.ds(off, 128)], idx_v[b])
            pltpu.async_copy(table_h.at[idx_v[b]], rows_v[b], sems[b])

        def consume(t, b):
            oo = os_h if b % 2 == 0 else oo_h
            off = pl.multiple_of(base + (t // 2) * 128, 8)
            pltpu.make_async_copy(
                table_h.at[idx_v[b]], rows_v[b], sems[b]).wait()
            pltpu.sync_copy(rows_v[b], oo.at[pl.ds(off, 128)])

        _pipelined_n(2 * nfull, 4, start, consume)

        if rem:
            off = pl.multiple_of(base + nfull * 128, 8)
            for ii_h, oo in ((is_h, os_h), (io_h, oo_h)):
                pltpu.sync_copy(ii_h.at[pl.ds(off, rem)], ir)
                pltpu.async_copy(table_h.at[ir], rr, sems[0]).wait()
                pltpu.sync_copy(rr, oo.at[pl.ds(off, rem)])

    return k(table, idx_s, idx_o)


def _sc_scatter_pool(new_s4, new_o4, s_idx, o_idx, zeros_stage, tag):
    neh = s_idx.shape[0]
    epw = neh // 16
    CK = 128
    nfull, rem = divmod(epw, CK)
    assert epw % 8 == 0 and rem % 8 == 0

    @functools.partial(
        pl.kernel,
        out_type=jax.ShapeDtypeStruct((NVP, H), _F32),
        mesh=_sc_mesh(),
        scratch_types=[
            pltpu.VMEM_SHARED((NVP, 128), _F32),
            pltpu.VMEM((CK,), _I32),
            pltpu.VMEM((max(rem, 8),), _I32),
            pltpu.VMEM((CK, 128), _F32),
            pltpu.VMEM((CK, 128), _F32),
            pltpu.VMEM((max(rem, 8), 128), _F32),
            pltpu.SemaphoreType.DMA,
            pltpu.SemaphoreType.DMA,
        ],
        name="sc_scatter_pool_" + tag,
    )
    def k(ns_h, no_h, si_h, oi_h, z_h, out_h,
          acc_s, idx_v, idx_r, vb0, vb1, val_r, sem0, sem1):
        cid = lax.axis_index("c")
        sid = lax.axis_index("s")
        ebase = pl.multiple_of(sid * epw, 8)
        rbase = pl.multiple_of(sid * (NVP // 16), 8)
        vbufs = (vb0, vb1)
        sems = (sem0, sem1)

        @pl.loop(0, 2)
        def _(jp):
            p = cid * 2 + jp
            col = p * 128
            pltpu.sync_copy(z_h, acc_s.at[pl.ds(rbase, NVP // 16)])
            plsc.subcore_barrier()

            def stream_edges(vals_h, ii_h):
                def start(j, b):
                    off = pl.multiple_of(ebase + j * CK, 8)
                    pltpu.async_copy(
                        vals_h.at[p, pl.ds(off, CK)], vbufs[b], sems[b])

                def consume(j, b):
                    off = pl.multiple_of(ebase + j * CK, 8)
                    pltpu.sync_copy(ii_h.at[pl.ds(off, CK)], idx_v)
                    pltpu.make_async_copy(
                        vals_h.at[p, pl.ds(off, CK)], vbufs[b],
                        sems[b]).wait()
                    pltpu.sync_copy(vbufs[b], acc_s.at[idx_v], add=True)

                _pipelined(nfull, start, consume)

            stream_edges(ns_h, si_h)
            stream_edges(no_h, oi_h)

            if rem:
                off = pl.multiple_of(ebase + nfull * CK, 8)
                for vals_h, ii_h in ((ns_h, si_h), (no_h, oi_h)):
                    pltpu.sync_copy(ii_h.at[pl.ds(off, rem)], idx_r)
                    pltpu.sync_copy(vals_h.at[p, pl.ds(off, rem)], val_r)
                    pltpu.sync_copy(val_r, acc_s.at[idx_r], add=True)

            plsc.subcore_barrier()
            pltpu.sync_copy(
                acc_s.at[pl.ds(rbase, NVP // 16)],
                out_h.at[pl.ds(rbase, NVP // 16), pl.ds(col, 128)])
            plsc.subcore_barrier()

    return k(new_s4, new_o4, s_idx, o_idx, zeros_stage)


def _sc_counts(s_idx_pad, o_idx_pad, zeros1):
    nep = s_idx_pad.shape[0]
    bpw = nep // NW
    nfull = bpw // 128
    assert bpw % 128 == 0
    rows = NVP // 16

    @functools.partial(
        pl.kernel,
        out_type=jax.ShapeDtypeStruct((2, NVP), _F32),
        mesh=_sc_mesh(),
        scratch_types=[
            pltpu.VMEM_SHARED((NVP,), _F32),
            pltpu.VMEM((128,), _I32),
            pltpu.VMEM((128,), _F32),
        ],
        name="sc_counts",
    )
    def k(si_h, oi_h, z_h, out_h, acc_s, idx_v, ones_v):
        cid = lax.axis_index("c")
        sid = lax.axis_index("s")
        wid = sid * 2 + cid
        base = pl.multiple_of(wid * bpw, 8)
        rbase = pl.multiple_of(sid * rows, 8)
        for q in range(8):
            ones_v[pl.ds(q * 16, 16)] = jnp.ones((16,), _F32)
        pltpu.sync_copy(z_h, acc_s.at[pl.ds(rbase, rows)])
        plsc.subcore_barrier()

        def accum(idx_h):
            @pl.loop(0, nfull)
            def _(j):
                off = pl.multiple_of(base + j * 128, 8)
                pltpu.sync_copy(idx_h.at[pl.ds(off, 128)], idx_v)
                pltpu.sync_copy(ones_v, acc_s.at[idx_v], add=True)

        accum(si_h)
        accum(oi_h)
        plsc.subcore_barrier()
        pltpu.sync_copy(acc_s.at[pl.ds(rbase, rows)],
                        out_h.at[cid, pl.ds(rbase, rows)])

    return k(s_idx_pad, o_idx_pad, zeros1)


def _full(shape):
    return pl.BlockSpec(shape, lambda i: (0, 0))


def _tc_embed(obj_rows, attr_rows, boxes8, ang_rows, w8, b):
    NB = 1024
    grid = NVP // NB

    def body(ob_r, at_r, bx_r, an_r, w_r, b_r, out_r):
        bv = jnp.dot(bx_r[...], w_r[...], preferred_element_type=_F32) + b_r[...]
        out_r[...] = jnp.concatenate(
            [ob_r[...][:, :OBJ_D], at_r[...][:, :ATTR_D], bv,
             an_r[...][:, :ANG_D]], axis=1).astype(jnp.bfloat16)

    return pl.pallas_call(
        body,
        grid=(grid,),
        in_specs=[
            pl.BlockSpec((NB, 128), lambda i: (i, 0)),
            pl.BlockSpec((NB, 128), lambda i: (i, 0)),
            pl.BlockSpec((NB, 8), lambda i: (i, 0)),
            pl.BlockSpec((NB, 128), lambda i: (i, 0)),
            _full((8, BOX_D)),
            _full((1, BOX_D)),
        ],
        out_specs=pl.BlockSpec((NB, GD), lambda i: (i, 0)),
        out_shape=jax.ShapeDtypeStruct((NVP, GD), jnp.bfloat16),
        name="tc_embed",
    )(obj_rows, attr_rows, boxes8, ang_rows, w8, b)


def _tc_edge_mlp(ovs, pv_or_pidx, ovo, ws, pred_tab_pad=None, want_p=True):
    EB = 1280
    neh = ovs.shape[0]
    grid = neh // EB
    (w1s, w1p, w1o, b1a, wbs, wbp, wbo, bbs, bbp, bbo) = ws
    layer0 = pred_tab_pad is not None

    def body(s_r, p_r, o_r, w1s_r, w1p_r, w1o_r, b1a_r,
             wbs_r, wbp_r, wbo_r, bbs_r, bbp_r, bbo_r,
             *rest):
        bf = jnp.bfloat16
        if layer0:
            pt_r = rest[0]
            rest = rest[1:]
        if want_p:
            os_r, op_r, oo_r = rest
        else:
            os_r, oo_r = rest
        if layer0:
            oh = (p_r[...] == lax.broadcasted_iota(
                _I32, (EB, pt_r.shape[0]), 1)).astype(bf)
            pv = jnp.dot(oh, pt_r[...].astype(bf), preferred_element_type=_F32)
        else:
            pv = p_r[...]
        def dot_packed(xi, w):
            lo = lax.bitcast_convert_type(xi << 16, _F32).astype(bf)
            hi = lax.bitcast_convert_type(
                xi & jnp.int32(-65536), _F32).astype(bf)
            return (jnp.dot(lo, w[:128].astype(bf), preferred_element_type=_F32)
                    + jnp.dot(hi, w[128:].astype(bf),
                              preferred_element_type=_F32))

        h = dot_packed(s_r[...], w1s_r[...]) + dot_packed(o_r[...], w1o_r[...])
        h = h + jnp.dot(pv.astype(bf), w1p_r[...].astype(bf),
                        preferred_element_type=_F32)
        h = jnp.maximum(h + b1a_r[...], 0.0).astype(bf)
        for kq in range(4):
            os_r[kq] = jnp.maximum(
                jnp.dot(h, wbs_r[...][:, kq * 128:(kq + 1) * 128].astype(bf),
                        preferred_element_type=_F32)
                + bbs_r[...][:, kq * 128:(kq + 1) * 128], 0.0)
            oo_r[kq] = jnp.maximum(
                jnp.dot(h, wbo_r[...][:, kq * 128:(kq + 1) * 128].astype(bf),
                        preferred_element_type=_F32)
                + bbo_r[...][:, kq * 128:(kq + 1) * 128], 0.0)
        if want_p:
            op_r[...] = jnp.maximum(
                jnp.dot(h, wbp_r[...].astype(bf), preferred_element_type=_F32)
                + bbp_r[...], 0.0).astype(bf)

    pspec = (pl.BlockSpec((EB, 1), lambda i: (i, 0)) if layer0
             else pl.BlockSpec((EB, GD), lambda i: (i, 0)))
    in_specs = [
        pl.BlockSpec((EB, 128), lambda i: (i, 0)),
        pspec,
        pl.BlockSpec((EB, 128), lambda i: (i, 0)),
        _full((GD, H)), _full((GD, H)), _full((GD, H)), _full((1, H)),
        _full((H, H)), _full((H, GD)), _full((H, H)),
        _full((1, H)), _full((1, GD)), _full((1, H)),
    ]
    args = [ovs, pv_or_pidx, ovo, w1s, w1p, w1o, b1a,
            wbs, wbp, wbo, bbs, bbp, bbo]
    if layer0:
        in_specs.append(_full(pred_tab_pad.shape))
        args.append(pred_tab_pad)
    return pl.pallas_call(
        body,
        grid=(grid,),
        in_specs=in_specs,
        out_specs=(
            [pl.BlockSpec((4, EB, 128), lambda i: (0, i, 0))]
            + ([pl.BlockSpec((EB, GD), lambda i: (i, 0))] if want_p else [])
            + [pl.BlockSpec((4, EB, 128), lambda i: (0, i, 0))]),
        out_shape=(
            [jax.ShapeDtypeStruct((4, neh, 128), _F32)]
            + ([jax.ShapeDtypeStruct((neh, GD), jnp.bfloat16)] if want_p
               else [])
            + [jax.ShapeDtypeStruct((4, neh, 128), _F32)]),
        name="tc_edge_mlp",
    )(*args)


def _tc_node_mlp(pooled_list, counts2, w2a, b2a, w2b, b2b):
    NB = 1024
    grid = NVP // NB
    np_parts = len(pooled_list)

    def body(*refs):
        p_refs = refs[:np_parts]
        c_r, w2a_r, b2a_r, w2b_r, b2b_r, out_r = refs[np_parts:]
        c = jnp.maximum(c_r[0, :] + c_r[1, :], 1.0)
        x = p_refs[0][...]
        for pr in p_refs[1:]:
            x = x + pr[...]
        x = x / c[:, None]
        hn = jnp.maximum(
            jnp.dot(x, w2a_r[...], preferred_element_type=_F32) + b2a_r[...], 0.0)
        out_r[...] = jnp.maximum(
            jnp.dot(hn, w2b_r[...], preferred_element_type=_F32) + b2b_r[...],
            0.0).astype(jnp.bfloat16)

    return pl.pallas_call(
        body,
        grid=(grid,),
        in_specs=(
            [pl.BlockSpec((NB, H), lambda i: (i, 0))] * np_parts
            + [pl.BlockSpec((2, NB), lambda i: (0, i)),
               _full((H, H)), _full((1, H)), _full((H, GD)), _full((1, GD))]),
        out_specs=pl.BlockSpec((NB, GD), lambda i: (i, 0)),
        out_shape=jax.ShapeDtypeStruct((NVP, GD), jnp.bfloat16),
        name="tc_node_mlp",
    )(*pooled_list, counts2, w2a, b2a, w2b, b2b)


def _tc_heads(ov, ws):
    NB = 1024
    grid = NVP // NB
    (bw1, bb1, bw2, bb2, aw1, ab1, aw2, ab2, bmw, avw_m, mu_b, bvw, avw_v, lv_b) = ws

    def body(x_r, bw1_r, bb1_r, bw2_r, bb2_r, aw1_r, ab1_r, aw2_r, ab2_r,
             bmw_r, awm_r, mub_r, bvw_r, awv_r, lvb_r, mu_r, lv_r):
        x = x_r[...].astype(_F32)
        hb = jnp.maximum(jnp.dot(x, bw1_r[...], preferred_element_type=_F32) + bb1_r[...], 0.0)
        hb = jnp.maximum(jnp.dot(hb, bw2_r[...], preferred_element_type=_F32) + bb2_r[...], 0.0)
        ha = jnp.maximum(jnp.dot(x, aw1_r[...], preferred_element_type=_F32) + ab1_r[...], 0.0)
        ha = jnp.maximum(jnp.dot(ha, aw2_r[...], preferred_element_type=_F32) + ab2_r[...], 0.0)
        mu_r[...] = (jnp.dot(hb, bmw_r[...], preferred_element_type=_F32)
                     + jnp.dot(ha, awm_r[...], preferred_element_type=_F32) + mub_r[...])
        lv_r[...] = (jnp.dot(hb, bvw_r[...], preferred_element_type=_F32)
                     + jnp.dot(ha, awv_r[...], preferred_element_type=_F32) + lvb_r[...])

    return pl.pallas_call(
        body,
        grid=(grid,),
        in_specs=[
            pl.BlockSpec((NB, GD), lambda i: (i, 0)),
            _full((GD, H)), _full((1, H)), _full((H, GD)), _full((1, GD)),
            _full((GD, H)), _full((1, H)), _full((H, GD)), _full((1, GD)),
            _full((GD, D)), _full((GD, D)), _full((1, D)),
            _full((GD, D)), _full((GD, D)), _full((1, D)),
        ],
        out_specs=[
            pl.BlockSpec((NB, D), lambda i: (i, 0)),
            pl.BlockSpec((NB, D), lambda i: (i, 0)),
        ],
        out_shape=[
            jax.ShapeDtypeStruct((NVP, D), _F32),
            jax.ShapeDtypeStruct((NVP, D), _F32),
        ],
        name="tc_heads",
    )(ov, *ws)


def kernel(objs, triples, boxes_gt, angles_gt, attributes, params):
    s_idx = triples[:, 0].astype(_I32)
    p_idx = triples[:, 1].astype(_I32)
    o_idx = triples[:, 2].astype(_I32)
    npad = NVP - NV
    objs_p = jnp.pad(objs.astype(_I32), (0, npad))
    attrs_p = jnp.pad(attributes.astype(_I32), (0, npad))
    angs_p = jnp.pad(angles_gt.astype(_I32), (0, npad))
    boxes8 = jnp.pad(boxes_gt, ((0, npad), (0, 2)))

    obj_rows = _sc_gather(
        jnp.pad(params['obj_tab'], ((0, 0), (0, 128 - OBJ_D))), objs_p,
        "sc_gather_obj")
    attr_rows = _sc_gather(
        jnp.pad(params['attr_tab'], ((0, 0), (0, 128 - ATTR_D))), attrs_p,
        "sc_gather_attr")
    ang_rows = _sc_gather(
        jnp.pad(params['angle_tab'], ((0, 0), (0, 128 - ANG_D))), angs_p,
        "sc_gather_angle")
    w8 = jnp.pad(params['box_w'], ((0, 2), (0, 0)))
    obj_vecs = _tc_embed(obj_rows, attr_rows, boxes8, ang_rows, w8,
                         params['box_b'].reshape(1, -1))

    nep = ((NE + NW * 128 - 1) // (NW * 128)) * NW * 128
    si_pad = jnp.pad(s_idx, (0, nep - NE), constant_values=NVP - 1)
    oi_pad = jnp.pad(o_idx, (0, nep - NE), constant_values=NVP - 1)
    counts2 = _sc_counts(si_pad, oi_pad, jnp.zeros((NVP // 16,), _F32))
    zeros_stage = jnp.zeros((NVP // 16, 128), _F32)

    halves = tuple((a, a + 32000) for a in range(0, NE, 32000))
    pred_vecs = [p_idx.reshape(NE, 1)[a:b] for a, b in halves]
    pred_tab_pad = jnp.pad(params['pred_tab'], ((0, 32 - 26), (0, 0)))

    def _perm(w):
        return jnp.concatenate([w[0::2], w[1::2]], axis=0)

    for li, lp in enumerate(params['gconv']):
        ws = (
            _perm(lp['w1a'][:GD]), lp['w1a'][GD:2 * GD],
            _perm(lp['w1a'][2 * GD:]),
            lp['b1a'].reshape(1, -1),
            lp['w1b'][:, :H], lp['w1b'][:, H:H + GD], lp['w1b'][:, H + GD:],
            lp['b1b'][:H].reshape(1, -1), lp['b1b'][H:H + GD].reshape(1, -1),
            lp['b1b'][H + GD:].reshape(1, -1),
        )
        objw = lax.bitcast_convert_type(obj_vecs.reshape(NVP, 128, 2), _I32)
        want_p = li < len(params['gconv']) - 1
        gathered = []
        pooled = [None] * len(halves)
        new_pv = [None] * len(halves)
        for hi, (a, b) in enumerate(halves):
            gathered.append(_sc_gather2(objw, s_idx[a:b], o_idx[a:b],
                                        f"sc_gather_so{hi}"))
        for hi, (a, b) in enumerate(halves):
            outs = _tc_edge_mlp(
                gathered[hi][0], pred_vecs[hi], gathered[hi][1], ws,
                pred_tab_pad=pred_tab_pad if li == 0 else None,
                want_p=want_p)
            if want_p:
                new_s, new_pv[hi], new_o = outs
            else:
                new_s, new_o = outs
            pooled[hi] = _sc_scatter_pool(
                new_s, new_o, s_idx[a:b], o_idx[a:b],
                zeros_stage, f"h{hi}")
        obj_vecs = _tc_node_mlp(pooled, counts2,
                                lp['w2a'], lp['b2a'].reshape(1, -1),
                                lp['w2b'], lp['b2b'].reshape(1, -1))
        pred_vecs = new_pv

    hws = (
        params['bmv_w1'], params['bmv_b1'].reshape(1, -1),
        params['bmv_w2'], params['bmv_b2'].reshape(1, -1),
        params['amv_w1'], params['amv_b1'].reshape(1, -1),
        params['amv_w2'], params['amv_b2'].reshape(1, -1),
        jnp.pad(params['bm_w'], ((0, 0), (0, D - BOX_D))),
        jnp.pad(params['am_w'], ((0, 0), (BOX_D, 0))),
        jnp.concatenate([params['bm_b'], params['am_b']]).reshape(1, -1),
        jnp.pad(params['bv_w'], ((0, 0), (0, D - BOX_D))),
        jnp.pad(params['av_w'], ((0, 0), (BOX_D, 0))),
        jnp.concatenate([params['bv_b'], params['av_b']]).reshape(1, -1),
    )
    mu, logvar = _tc_heads(obj_vecs, hws)
    return mu[:NV], logvar[:NV]


# speedup vs baseline: 1.0549x; 1.0549x over previous
import functools

import jax
import jax.numpy as jnp
from jax import lax
from jax.experimental import pallas as pl
from jax.experimental.pallas import tpu as pltpu
from jax.experimental.pallas import tpu_sc as plsc

D = 128
OBJ_D = 96
ATTR_D = 32
ANG_D = 32
BOX_D = 96
GD = 2 * D
H = 4 * D
NANGLE = 24
NV = 10000
NVP = 10240
NE = 160000
NW = 32

_F32 = jnp.float32
_I32 = jnp.int32


def _sc_mesh():
    return plsc.VectorSubcoreMesh(
        core_axis_name="c", subcore_axis_name="s", num_cores=2, num_subcores=16)


def _pipelined_n(ntasks, nbuf, start, consume):
    for t in range(min(nbuf - 1, ntasks)):
        start(t, t % nbuf)
    ngroups = ntasks // nbuf
    if ngroups:
        @pl.loop(0, nbuf * ngroups, step=nbuf)
        def _(j):
            for b in range(nbuf):
                t = j + b

                @pl.when(t + nbuf - 1 < ntasks)
                def _():
                    start(t + nbuf - 1, (b + nbuf - 1) % nbuf)

                consume(t, b)
    for t in range(nbuf * ngroups, ntasks):
        consume(t, t % nbuf)


def _pipelined(nfull, start, consume):
    if nfull <= 0:
        return
    start(0, 0)
    npairs = nfull // 2
    if npairs:
        @pl.loop(0, 2 * npairs, step=2)
        def _(j):
            for b in range(2):
                jj = j + b

                @pl.when(jj + 1 < nfull)
                def _():
                    start(jj + 1, 1 - b)

                consume(jj, b)
    if nfull % 2:
        consume(nfull - 1, (nfull - 1) % 2)


def _gather_stream(table_h, idx_h, out_h, bufs, base, nfull, rem):
    (idx_v, rows_v, sems) = bufs

    def start(j, b):
        off = pl.multiple_of(base + j * 128, 8)
        pltpu.sync_copy(idx_h.at[pl.ds(off, 128)], idx_v[b])
        pltpu.async_copy(table_h.at[idx_v[b]], rows_v[b], sems[b])

    def consume(j, b):
        off = pl.multiple_of(base + j * 128, 8)
        pltpu.make_async_copy(
            table_h.at[idx_v[b]], rows_v[b], sems[b]).wait()
        pltpu.sync_copy(rows_v[b], out_h.at[pl.ds(off, 128)])

    _pipelined(nfull, start, consume)

    if rem:
        off = pl.multiple_of(base + nfull * 128, 8)
        pltpu.sync_copy(idx_h.at[pl.ds(off, rem)], idx_v[2])
        pltpu.async_copy(table_h.at[idx_v[2]], rows_v[2], sems[0]).wait()
        pltpu.sync_copy(rows_v[2], out_h.at[pl.ds(off, rem)])


def _gather_scratch(row_shape, dtype, rem):
    sc = [
        pltpu.VMEM((128,), _I32), pltpu.VMEM((128,), _I32),
        pltpu.VMEM((max(rem, 8),), _I32),
        pltpu.VMEM((128,) + row_shape, dtype),
        pltpu.VMEM((128,) + row_shape, dtype),
        pltpu.VMEM((max(rem, 8),) + row_shape, dtype),
        pltpu.SemaphoreType.DMA, pltpu.SemaphoreType.DMA,
    ]
    return sc


def _sc_gather(table, idx, name):
    B = idx.shape[0]
    row_shape = table.shape[1:]
    bpw = B // NW
    assert B % NW == 0 and bpw % 8 == 0
    nfull, rem = divmod(bpw, 128)

    @functools.partial(
        pl.kernel,
        out_type=jax.ShapeDtypeStruct((B,) + row_shape, table.dtype),
        mesh=_sc_mesh(),
        scratch_types=_gather_scratch(row_shape, table.dtype, rem),
        name=name,
    )
    def k(table_h, idx_h, out_h, i0, i1, i2, r0, r1, r2, s0, s1):
        wid = lax.axis_index("s") * 2 + lax.axis_index("c")
        base = pl.multiple_of(wid * bpw, 8)
        bufs = ((i0, i1, i2), (r0, r1, r2), (s0, s1))
        _gather_stream(table_h, idx_h, out_h, bufs, base, nfull, rem)

    return k(table, idx)


def _sc_gather2(table, idx_s, idx_o, name):
    B = idx_s.shape[0]
    row_shape = table.shape[1:]
    bpw = B // NW
    assert B % NW == 0 and bpw % 8 == 0
    nfull, rem = divmod(bpw, 128)
    out = jax.ShapeDtypeStruct((B,) + row_shape, table.dtype)
    scratch = (
        [pltpu.VMEM((128,), _I32)] * 4
        + [pltpu.VMEM((max(rem, 8),), _I32)]
        + [pltpu.VMEM((128,) + row_shape, table.dtype)] * 4
        + [pltpu.VMEM((max(rem, 8),) + row_shape, table.dtype)]
        + [pltpu.SemaphoreType.DMA] * 4)

    @functools.partial(
        pl.kernel,
        out_type=[out, out],
        mesh=_sc_mesh(),
        scratch_types=scratch,
        name=name,
    )
    def k(table_h, is_h, io_h, os_h, oo_h,
          i0, i1, i2, i3, ir, r0, r1, r2, r3, rr, s0, s1, s2, s3):
        wid = lax.axis_index("s") * 2 + lax.axis_index("c")
        base = pl.multiple_of(wid * bpw, 8)
        idx_v = (i0, i1, i2, i3)
        rows_v = (r0, r1, r2, r3)
        sems = (s0, s1, s2, s3)

        def start(t, b):
            ii_h = is_h if b % 2 == 0 else io_h
            off = pl.multiple_of(base + (t // 2) * 128, 8)
            pltpu.sync_copy(ii_h.at[pl.ds(off, 128)], idx_v[b])
            pltpu.async_copy(table_h.at[idx_v[b]], rows_v[b], sems[b])

        def consume(t, b):
            oo = os_h if b % 2 == 0 else oo_h
            off = pl.multiple_of(base + (t // 2) * 128, 8)
            pltpu.make_async_copy(
                table_h.at[idx_v[b]], rows_v[b], sems[b]).wait()
            pltpu.sync_copy(rows_v[b], oo.at[pl.ds(off, 128)])

        _pipelined_n(2 * nfull, 4, start, consume)

        if rem:
            off = pl.multiple_of(base + nfull * 128, 8)
            for ii_h, oo in ((is_h, os_h), (io_h, oo_h)):
                pltpu.sync_copy(ii_h.at[pl.ds(off, rem)], ir)
                pltpu.async_copy(table_h.at[ir], rr, sems[0]).wait()
                pltpu.sync_copy(rr, oo.at[pl.ds(off, rem)])

    return k(table, idx_s, idx_o)


def _sc_scatter_pool(new_s4, new_o4, s_idx, o_idx, zeros_stage, tag):
    neh = s_idx.shape[0]
    epw = neh // 16
    CK = 128
    nfull, rem = divmod(epw, CK)
    assert epw % 8 == 0 and rem % 8 == 0

    @functools.partial(
        pl.kernel,
        out_type=jax.ShapeDtypeStruct((NVP, H), _F32),
        mesh=_sc_mesh(),
        scratch_types=[
            pltpu.VMEM_SHARED((NVP, 128), _F32),
            pltpu.VMEM((CK,), _I32),
            pltpu.VMEM((max(rem, 8),), _I32),
            pltpu.VMEM((CK, 128), _F32),
            pltpu.VMEM((CK, 128), _F32),
            pltpu.VMEM((max(rem, 8), 128), _F32),
            pltpu.SemaphoreType.DMA,
            pltpu.SemaphoreType.DMA,
        ],
        name="sc_scatter_pool_" + tag,
    )
    def k(ns_h, no_h, si_h, oi_h, z_h, out_h,
          acc_s, idx_v, idx_r, vb0, vb1, val_r, sem0, sem1):
        cid = lax.axis_index("c")
        sid = lax.axis_index("s")
        ebase = pl.multiple_of(sid * epw, 8)
        rbase = pl.multiple_of(sid * (NVP // 16), 8)
        vbufs = (vb0, vb1)
        sems = (sem0, sem1)

        @pl.loop(0, 2)
        def _(jp):
            p = cid * 2 + jp
            col = p * 128
            pltpu.sync_copy(z_h, acc_s.at[pl.ds(rbase, NVP // 16)])
            plsc.subcore_barrier()

            def stream_edges(vals_h, ii_h):
                def start(j, b):
                    off = pl.multiple_of(ebase + j * CK, 8)
                    pltpu.async_copy(
                        vals_h.at[p, pl.ds(off, CK)], vbufs[b], sems[b])

                def consume(j, b):
                    off = pl.multiple_of(ebase + j * CK, 8)
                    pltpu.sync_copy(ii_h.at[pl.ds(off, CK)], idx_v)
                    pltpu.make_async_copy(
                        vals_h.at[p, pl.ds(off, CK)], vbufs[b],
                        sems[b]).wait()
                    pltpu.sync_copy(vbufs[b], acc_s.at[idx_v], add=True)

                _pipelined(nfull, start, consume)

            stream_edges(ns_h, si_h)
            stream_edges(no_h, oi_h)

            if rem:
                off = pl.multiple_of(ebase + nfull * CK, 8)
                for vals_h, ii_h in ((ns_h, si_h), (no_h, oi_h)):
                    pltpu.sync_copy(ii_h.at[pl.ds(off, rem)], idx_r)
                    pltpu.sync_copy(vals_h.at[p, pl.ds(off, rem)], val_r)
                    pltpu.sync_copy(val_r, acc_s.at[idx_r], add=True)

            plsc.subcore_barrier()
            pltpu.sync_copy(
                acc_s.at[pl.ds(rbase, NVP // 16)],
                out_h.at[pl.ds(rbase, NVP // 16), pl.ds(col, 128)])
            plsc.subcore_barrier()

    return k(new_s4, new_o4, s_idx, o_idx, zeros_stage)


def _sc_counts(s_idx_pad, o_idx_pad, zeros1):
    nep = s_idx_pad.shape[0]
    bpw = nep // NW
    nfull = bpw // 128
    assert bpw % 128 == 0
    rows = NVP // 16

    @functools.partial(
        pl.kernel,
        out_type=jax.ShapeDtypeStruct((2, NVP), _F32),
        mesh=_sc_mesh(),
        scratch_types=[
            pltpu.VMEM_SHARED((NVP,), _F32),
            pltpu.VMEM((128,), _I32),
            pltpu.VMEM((128,), _F32),
        ],
        name="sc_counts",
    )
    def k(si_h, oi_h, z_h, out_h, acc_s, idx_v, ones_v):
        cid = lax.axis_index("c")
        sid = lax.axis_index("s")
        wid = sid * 2 + cid
        base = pl.multiple_of(wid * bpw, 8)
        rbase = pl.multiple_of(sid * rows, 8)
        for q in range(8):
            ones_v[pl.ds(q * 16, 16)] = jnp.ones((16,), _F32)
        pltpu.sync_copy(z_h, acc_s.at[pl.ds(rbase, rows)])
        plsc.subcore_barrier()

        def accum(idx_h):
            @pl.loop(0, nfull)
            def _(j):
                off = pl.multiple_of(base + j * 128, 8)
                pltpu.sync_copy(idx_h.at[pl.ds(off, 128)], idx_v)
                pltpu.sync_copy(ones_v, acc_s.at[idx_v], add=True)

        accum(si_h)
        accum(oi_h)
        plsc.subcore_barrier()
        pltpu.sync_copy(acc_s.at[pl.ds(rbase, rows)],
                        out_h.at[cid, pl.ds(rbase, rows)])

    return k(s_idx_pad, o_idx_pad, zeros1)


def _full(shape):
    return pl.BlockSpec(shape, lambda i: (0, 0))


def _tc_embed(obj_rows, attr_rows, boxes8, ang_rows, w8, b):
    NB = 1024
    grid = NVP // NB

    def body(ob_r, at_r, bx_r, an_r, w_r, b_r, out_r):
        bv = jnp.dot(bx_r[...], w_r[...], preferred_element_type=_F32) + b_r[...]
        out_r[...] = jnp.concatenate(
            [ob_r[...][:, :OBJ_D], at_r[...][:, :ATTR_D], bv,
             an_r[...][:, :ANG_D]], axis=1).astype(jnp.bfloat16)

    return pl.pallas_call(
        body,
        grid=(grid,),
        in_specs=[
            pl.BlockSpec((NB, 128), lambda i: (i, 0)),
            pl.BlockSpec((NB, 128), lambda i: (i, 0)),
            pl.BlockSpec((NB, 8), lambda i: (i, 0)),
            pl.BlockSpec((NB, 128), lambda i: (i, 0)),
            _full((8, BOX_D)),
            _full((1, BOX_D)),
        ],
        out_specs=pl.BlockSpec((NB, GD), lambda i: (i, 0)),
        out_shape=jax.ShapeDtypeStruct((NVP, GD), jnp.bfloat16),
        name="tc_embed",
    )(obj_rows, attr_rows, boxes8, ang_rows, w8, b)


def _tc_edge_mlp(ovs, pv_or_pidx, ovo, ws, pred_tab_pad=None, want_p=True):
    EB = 1280
    neh = ovs.shape[0]
    grid = neh // EB
    (w1s, w1p, w1o, b1a, wbs, wbp, wbo, bbs, bbp, bbo) = ws
    layer0 = pred_tab_pad is not None

    def body(s_r, p_r, o_r, w1s_r, w1p_r, w1o_r, b1a_r,
             wbs_r, wbp_r, wbo_r, bbs_r, bbp_r, bbo_r,
             *rest):
        bf = jnp.bfloat16
        if layer0:
            pt_r = rest[0]
            rest = rest[1:]
        if want_p:
            os_r, op_r, oo_r = rest
        else:
            os_r, oo_r = rest
        if layer0:
            oh = (p_r[...] == lax.broadcasted_iota(
                _I32, (EB, pt_r.shape[0]), 1)).astype(bf)
            pv = jnp.dot(oh, pt_r[...].astype(bf), preferred_element_type=_F32)
        else:
            pv = p_r[...]
        def dot_packed(xi, w):
            lo = lax.bitcast_convert_type(xi << 16, _F32).astype(bf)
            hi = lax.bitcast_convert_type(
                xi & jnp.int32(-65536), _F32).astype(bf)
            return (jnp.dot(lo, w[:128].astype(bf), preferred_element_type=_F32)
                    + jnp.dot(hi, w[128:].astype(bf),
                              preferred_element_type=_F32))

        h = dot_packed(s_r[...], w1s_r[...]) + dot_packed(o_r[...], w1o_r[...])
        h = h + jnp.dot(pv.astype(bf), w1p_r[...].astype(bf),
                        preferred_element_type=_F32)
        h = jnp.maximum(h + b1a_r[...], 0.0).astype(bf)
        for kq in range(4):
            os_r[kq] = jnp.maximum(
                jnp.dot(h, wbs_r[...][:, kq * 128:(kq + 1) * 128].astype(bf),
                        preferred_element_type=_F32)
                + bbs_r[...][:, kq * 128:(kq + 1) * 128], 0.0)
            oo_r[kq] = jnp.maximum(
                jnp.dot(h, wbo_r[...][:, kq * 128:(kq + 1) * 128].astype(bf),
                        preferred_element_type=_F32)
                + bbo_r[...][:, kq * 128:(kq + 1) * 128], 0.0)
        if want_p:
            op_r[...] = jnp.maximum(
                jnp.dot(h, wbp_r[...].astype(bf), preferred_element_type=_F32)
                + bbp_r[...], 0.0).astype(bf)

    pspec = (pl.BlockSpec((EB, 1), lambda i: (i, 0)) if layer0
             else pl.BlockSpec((EB, GD), lambda i: (i, 0)))
    in_specs = [
        pl.BlockSpec((EB, 128), lambda i: (i, 0)),
        pspec,
        pl.BlockSpec((EB, 128), lambda i: (i, 0)),
        _full((GD, H)), _full((GD, H)), _full((GD, H)), _full((1, H)),
        _full((H, H)), _full((H, GD)), _full((H, H)),
        _full((1, H)), _full((1, GD)), _full((1, H)),
    ]
    args = [ovs, pv_or_pidx, ovo, w1s, w1p, w1o, b1a,
            wbs, wbp, wbo, bbs, bbp, bbo]
    if layer0:
        in_specs.append(_full(pred_tab_pad.shape))
        args.append(pred_tab_pad)
    return pl.pallas_call(
        body,
        grid=(grid,),
        in_specs=in_specs,
        out_specs=(
            [pl.BlockSpec((4, EB, 128), lambda i: (0, i, 0))]
            + ([pl.BlockSpec((EB, GD), lambda i: (i, 0))] if want_p else [])
            + [pl.BlockSpec((4, EB, 128), lambda i: (0, i, 0))]),
        out_shape=(
            [jax.ShapeDtypeStruct((4, neh, 128), _F32)]
            + ([jax.ShapeDtypeStruct((neh, GD), jnp.bfloat16)] if want_p
               else [])
            + [jax.ShapeDtypeStruct((4, neh, 128), _F32)]),
        name="tc_edge_mlp",
    )(*args)


def _tc_node_mlp(pooled_list, counts2, w2a, b2a, w2b, b2b):
    NB = 1024
    grid = NVP // NB
    np_parts = len(pooled_list)

    def body(*refs):
        p_refs = refs[:np_parts]
        c_r, w2a_r, b2a_r, w2b_r, b2b_r, out_r = refs[np_parts:]
        c = jnp.maximum(c_r[0, :] + c_r[1, :], 1.0)
        x = p_refs[0][...]
        for pr in p_refs[1:]:
            x = x + pr[...]
        x = x / c[:, None]
        hn = jnp.maximum(
            jnp.dot(x, w2a_r[...], preferred_element_type=_F32) + b2a_r[...], 0.0)
        out_r[...] = jnp.maximum(
            jnp.dot(hn, w2b_r[...], preferred_element_type=_F32) + b2b_r[...],
            0.0).astype(jnp.bfloat16)

    return pl.pallas_call(
        body,
        grid=(grid,),
        in_specs=(
            [pl.BlockSpec((NB, H), lambda i: (i, 0))] * np_parts
            + [pl.BlockSpec((2, NB), lambda i: (0, i)),
               _full((H, H)), _full((1, H)), _full((H, GD)), _full((1, GD))]),
        out_specs=pl.BlockSpec((NB, GD), lambda i: (i, 0)),
        out_shape=jax.ShapeDtypeStruct((NVP, GD), jnp.bfloat16),
        name="tc_node_mlp",
    )(*pooled_list, counts2, w2a, b2a, w2b, b2b)


def _tc_heads(ov, ws):
    NB = 1024
    grid = NVP // NB
    (bw1, bb1, bw2, bb2, aw1, ab1, aw2, ab2, bmw, avw_m, mu_b, bvw, avw_v, lv_b) = ws

    def body(x_r, bw1_r, bb1_r, bw2_r, bb2_r, aw1_r, ab1_r, aw2_r, ab2_r,
             bmw_r, awm_r, mub_r, bvw_r, awv_r, lvb_r, mu_r, lv_r):
        x = x_r[...].astype(_F32)
        hb = jnp.maximum(jnp.dot(x, bw1_r[...], preferred_element_type=_F32) + bb1_r[...], 0.0)
        hb = jnp.maximum(jnp.dot(hb, bw2_r[...], preferred_element_type=_F32) + bb2_r[...], 0.0)
        ha = jnp.maximum(jnp.dot(x, aw1_r[...], preferred_element_type=_F32) + ab1_r[...], 0.0)
        ha = jnp.maximum(jnp.dot(ha, aw2_r[...], preferred_element_type=_F32) + ab2_r[...], 0.0)
        mu_r[...] = (jnp.dot(hb, bmw_r[...], preferred_element_type=_F32)
                     + jnp.dot(ha, awm_r[...], preferred_element_type=_F32) + mub_r[...])
        lv_r[...] = (jnp.dot(hb, bvw_r[...], preferred_element_type=_F32)
                     + jnp.dot(ha, awv_r[...], preferred_element_type=_F32) + lvb_r[...])

    return pl.pallas_call(
        body,
        grid=(grid,),
        in_specs=[
            pl.BlockSpec((NB, GD), lambda i: (i, 0)),
            _full((GD, H)), _full((1, H)), _full((H, GD)), _full((1, GD)),
            _full((GD, H)), _full((1, H)), _full((H, GD)), _full((1, GD)),
            _full((GD, D)), _full((GD, D)), _full((1, D)),
            _full((GD, D)), _full((GD, D)), _full((1, D)),
        ],
        out_specs=[
            pl.BlockSpec((NB, D), lambda i: (i, 0)),
            pl.BlockSpec((NB, D), lambda i: (i, 0)),
        ],
        out_shape=[
            jax.ShapeDtypeStruct((NVP, D), _F32),
            jax.ShapeDtypeStruct((NVP, D), _F32),
        ],
        name="tc_heads",
    )(ov, *ws)


def kernel(objs, triples, boxes_gt, angles_gt, attributes, params):
    s_idx = triples[:, 0].astype(_I32)
    p_idx = triples[:, 1].astype(_I32)
    o_idx = triples[:, 2].astype(_I32)
    npad = NVP - NV
    objs_p = jnp.pad(objs.astype(_I32), (0, npad))
    attrs_p = jnp.pad(attributes.astype(_I32), (0, npad))
    angs_p = jnp.pad(angles_gt.astype(_I32), (0, npad))
    boxes8 = jnp.pad(boxes_gt, ((0, npad), (0, 2)))

    obj_rows = _sc_gather(
        jnp.pad(params['obj_tab'], ((0, 0), (0, 128 - OBJ_D))), objs_p,
        "sc_gather_obj")
    attr_rows = _sc_gather(
        jnp.pad(params['attr_tab'], ((0, 0), (0, 128 - ATTR_D))), attrs_p,
        "sc_gather_attr")
    ang_rows = _sc_gather(
        jnp.pad(params['angle_tab'], ((0, 0), (0, 128 - ANG_D))), angs_p,
        "sc_gather_angle")
    w8 = jnp.pad(params['box_w'], ((0, 2), (0, 0)))
    obj_vecs = _tc_embed(obj_rows, attr_rows, boxes8, ang_rows, w8,
                         params['box_b'].reshape(1, -1))

    nep = ((NE + NW * 128 - 1) // (NW * 128)) * NW * 128
    si_pad = jnp.pad(s_idx, (0, nep - NE), constant_values=NVP - 1)
    oi_pad = jnp.pad(o_idx, (0, nep - NE), constant_values=NVP - 1)
    counts2 = _sc_counts(si_pad, oi_pad, jnp.zeros((NVP // 16,), _F32))
    zeros_stage = jnp.zeros((NVP // 16, 128), _F32)

    halves = ((0, 40960), (40960, 81920), (81920, 121600), (121600, NE))
    pred_vecs = [p_idx.reshape(NE, 1)[a:b] for a, b in halves]
    pred_tab_pad = jnp.pad(params['pred_tab'], ((0, 32 - 26), (0, 0)))

    def _perm(w):
        return jnp.concatenate([w[0::2], w[1::2]], axis=0)

    for li, lp in enumerate(params['gconv']):
        ws = (
            _perm(lp['w1a'][:GD]), lp['w1a'][GD:2 * GD],
            _perm(lp['w1a'][2 * GD:]),
            lp['b1a'].reshape(1, -1),
            lp['w1b'][:, :H], lp['w1b'][:, H:H + GD], lp['w1b'][:, H + GD:],
            lp['b1b'][:H].reshape(1, -1), lp['b1b'][H:H + GD].reshape(1, -1),
            lp['b1b'][H + GD:].reshape(1, -1),
        )
        objw = lax.bitcast_convert_type(obj_vecs.reshape(NVP, 128, 2), _I32)
        want_p = li < len(params['gconv']) - 1
        gathered = []
        pooled = [None] * len(halves)
        new_pv = [None] * len(halves)
        for hi, (a, b) in enumerate(halves):
            gathered.append(_sc_gather2(objw, s_idx[a:b], o_idx[a:b],
                                        f"sc_gather_so{hi}"))
        for hi, (a, b) in enumerate(halves):
            outs = _tc_edge_mlp(
                gathered[hi][0], pred_vecs[hi], gathered[hi][1], ws,
                pred_tab_pad=pred_tab_pad if li == 0 else None,
                want_p=want_p)
            if want_p:
                new_s, new_pv[hi], new_o = outs
            else:
                new_s, new_o = outs
            pooled[hi] = _sc_scatter_pool(
                new_s, new_o, s_idx[a:b], o_idx[a:b],
                zeros_stage, f"h{hi}")
        obj_vecs = _tc_node_mlp(pooled, counts2,
                                lp['w2a'], lp['b2a'].reshape(1, -1),
                                lp['w2b'], lp['b2b'].reshape(1, -1))
        pred_vecs = new_pv

    hws = (
        params['bmv_w1'], params['bmv_b1'].reshape(1, -1),
        params['bmv_w2'], params['bmv_b2'].reshape(1, -1),
        params['amv_w1'], params['amv_b1'].reshape(1, -1),
        params['amv_w2'], params['amv_b2'].reshape(1, -1),
        jnp.pad(params['bm_w'], ((0, 0), (0, D - BOX_D))),
        jnp.pad(params['am_w'], ((0, 0), (BOX_D, 0))),
        jnp.concatenate([params['bm_b'], params['am_b']]).reshape(1, -1),
        jnp.pad(params['bv_w'], ((0, 0), (0, D - BOX_D))),
        jnp.pad(params['av_w'], ((0, 0), (BOX_D, 0))),
        jnp.concatenate([params['bv_b'], params['av_b']]).reshape(1, -1),
    )
    mu, logvar = _tc_heads(obj_vecs, hws)
    return mu[:NV], logvar[:NV]
